# baseline (speedup 1.0000x reference)
import functools

import jax
import jax.numpy as jnp
from jax import lax
from jax.experimental import pallas as pl
from jax.experimental.pallas import tpu as pltpu
from jax.experimental.pallas import tpu_sc as plsc

L = 16


def _fedfast_sc(user, item, user_table, item_table, aff_w, aff_b):
    B = user.shape[0]
    F = user_table.shape[1]
    info = plsc.get_sparse_core_info()
    NC, NS = info.num_cores, info.num_subcores
    NW = NC * NS
    b_per_w = B // NW
    CHUNK = 128
    n_chunks = b_per_w // CHUNK
    n_f = F // L

    mesh = plsc.VectorSubcoreMesh(core_axis_name="c", subcore_axis_name="s")

    @functools.partial(
        pl.kernel,
        mesh=mesh,
        compiler_params=pltpu.CompilerParams(needs_layout_passes=False),
        out_type=jax.ShapeDtypeStruct((B,), jnp.float32),
        scratch_types=[
            pltpu.VMEM((b_per_w,), jnp.int32),
            pltpu.VMEM((b_per_w,), jnp.int32),
            pltpu.VMEM((CHUNK, F), jnp.float32),
            pltpu.VMEM((CHUNK, F), jnp.float32),
            pltpu.VMEM((F,), jnp.float32),
            pltpu.VMEM((L,), jnp.float32),
            pltpu.VMEM((b_per_w,), jnp.float32),
            pltpu.SemaphoreType.DMA,
            pltpu.SemaphoreType.DMA,
        ],
    )
    def k(user_hbm, item_hbm, ut_hbm, it_hbm, w_hbm, b_hbm, out_hbm,
          uidx_v, iidx_v, u_v, v_v, w_v, b_v, out_v, sem_u, sem_v):
        wid = lax.axis_index("s") * NC + lax.axis_index("c")
        base = wid * b_per_w

        pltpu.sync_copy(user_hbm.at[pl.ds(base, b_per_w)], uidx_v)
        pltpu.sync_copy(item_hbm.at[pl.ds(base, b_per_w)], iidx_v)
        pltpu.sync_copy(w_hbm.at[0], w_v)
        pltpu.sync_copy(b_hbm, b_v)

        wr = [w_v[pl.ds(i * L, L)] for i in range(n_f)]
        bias = b_v[...]
        lane = lax.iota(jnp.int32, L)

        for c in range(n_chunks):
            cu = pltpu.async_copy(
                ut_hbm.at[uidx_v.at[pl.ds(c * CHUNK, CHUNK)]], u_v, sem_u)
            cv = pltpu.async_copy(
                it_hbm.at[iidx_v.at[pl.ds(c * CHUNK, CHUNK)]], v_v, sem_v)
            cu.wait()
            cv.wait()

            def grp(g, _, c=c):
                out_vec = jnp.zeros((L,), jnp.float32)
                for j in range(L):
                    r = g * L + j
                    acc = jnp.zeros((L,), jnp.float32)
                    for f in range(n_f):
                        acc = acc + (u_v[r, pl.ds(f * L, L)]
                                     * v_v[r, pl.ds(f * L, L)]) * wr[f]
                    s = jnp.sum(acc)
                    out_vec = jnp.where(lane == j, s, out_vec)
                out_v[pl.ds(c * CHUNK + g * L, L)] = out_vec + bias
                return 0

            lax.fori_loop(0, CHUNK // L, grp, 0)

        pltpu.sync_copy(out_v, out_hbm.at[pl.ds(base, b_per_w)])

    bias_vec = jnp.broadcast_to(aff_b.astype(jnp.float32), (L,))
    return k(user, item, user_table, item_table, aff_w, bias_vec)


def kernel(user, item, user_table, item_table, aff_W, aff_b):
    return _fedfast_sc(user, item, user_table, item_table, aff_W, aff_b)
